# baseline (speedup 1.0000x reference)
import functools

import jax
import jax.numpy as jnp
from jax import lax
from jax.experimental import pallas as pl
from jax.experimental.pallas import tpu as pltpu
from jax.experimental.pallas import tpu_sc as plsc

NUM_HEADS = 8
OUT_SIZE = 16
DH = OUT_SIZE * NUM_HEADS

NC = 2
NS = 16
NW = NC * NS
LANES = 16
C = 128
ACC_W = 144


def _stage1_body(z_ref, wt_ref, b_ref, alw_ref, arw_ref,
                 tcol_ref, lrow_ref, lcol_ref, g_ref):
    n = z_ref.shape[0]
    pad = tcol_ref.shape[0] - n
    h = jnp.dot(z_ref[...], wt_ref[...], preferred_element_type=jnp.float32,
                 precision=jax.lax.Precision.HIGHEST)
    h = h + b_ref[...]
    el = jnp.dot(h, alw_ref[...], preferred_element_type=jnp.float32,
                 precision=jax.lax.Precision.HIGHEST)
    er = jnp.dot(h, arw_ref[...], preferred_element_type=jnp.float32,
                 precision=jax.lax.Precision.HIGHEST)
    zpad_h = jnp.zeros((pad, DH), jnp.float32)
    zpad_e = jnp.zeros((pad, NUM_HEADS), jnp.float32)
    tcol_ref[...] = jnp.concatenate([h, zpad_h], axis=0)
    elp = jnp.concatenate([el, zpad_e], axis=0)
    erp = jnp.concatenate([er, zpad_e], axis=0)
    lrow_ref[...] = jnp.concatenate([elp, elp], axis=1)
    lcol_ref[...] = jnp.concatenate([erp, erp], axis=1)
    gmax = jnp.max(el, axis=0) + jnp.max(er, axis=0)
    g = jnp.maximum(gmax, 0.01 * gmax)
    g_ref[...] = jnp.concatenate([g, g]).reshape(1, 2 * NUM_HEADS)


def _stage3_body(s2_ref, t_ref, o_ref):
    s = s2_ref[0] + s2_ref[1]
    msg = s[:, 0:DH]
    den = s[:, DH:DH + NUM_HEADS]
    r = 1.0 / den
    r128 = jnp.dot(r, t_ref[...], preferred_element_type=jnp.float32,
                 precision=jax.lax.Precision.HIGHEST)
    o_ref[...] = msg * r128


def _make_sc_edge_kernel(n_tab, n_acc, e_pad, k_steps):
    per_w = C * k_steps
    rows_per = n_acc // NS

    mesh = plsc.VectorSubcoreMesh(core_axis_name="c", subcore_axis_name="s")

    @functools.partial(
        pl.kernel,
        out_type=jax.ShapeDtypeStruct((NC, n_acc, ACC_W), jnp.float32),
        mesh=mesh,
        compiler_params=pltpu.CompilerParams(use_tc_tiling_on_sc=False),
        scratch_types=[
            pltpu.VMEM((C,), jnp.int32),
            pltpu.VMEM((C,), jnp.int32),
            pltpu.VMEM((C, 2 * NUM_HEADS), jnp.float32),
            pltpu.VMEM((C, 2 * NUM_HEADS), jnp.float32),
            pltpu.VMEM((C, DH), jnp.float32),
            pltpu.VMEM((C, ACC_W), jnp.float32),
            pltpu.VMEM((2 * NUM_HEADS,), jnp.float32),
            pltpu.VMEM_SHARED((n_acc, ACC_W), jnp.float32),
            pltpu.SemaphoreType.DMA,
            pltpu.SemaphoreType.DMA,
            pltpu.SemaphoreType.DMA,
        ],
    )
    def sc_edges(row_hbm, col_hbm, lrow_hbm, lcol_hbm, tcol_hbm, g_hbm,
                 out_hbm, idxr, idxc, ra, rb, hc, m, gv, acc, s1, s2, s3):
        ci = lax.axis_index("c")
        si = lax.axis_index("s")
        wid = ci * NS + si

        zvec = jnp.zeros((LANES,), jnp.float32)

        @pl.loop(0, C)
        def _(r):
            for j in range(ACC_W // LANES):
                m[r, pl.ds(j * LANES, LANES)] = zvec

        base_row = si * rows_per

        r0 = 0
        while r0 < rows_per:
            nr = min(rows_per - r0, C)
            pltpu.sync_copy(m.at[pl.ds(0, nr)],
                            acc.at[pl.ds(base_row + r0, nr)])
            r0 += nr

        pltpu.sync_copy(g_hbm.at[0], gv)
        plsc.subcore_barrier()

        gvec = gv[...]

        @pl.loop(0, k_steps)
        def _(step):
            base = wid * per_w + step * C
            pltpu.sync_copy(row_hbm.at[pl.ds(base, C)], idxr)
            pltpu.sync_copy(col_hbm.at[pl.ds(base, C)], idxc)
            cp1 = pltpu.async_copy(lrow_hbm.at[idxr], ra, s1)
            cp2 = pltpu.async_copy(lcol_hbm.at[idxc], rb, s2)
            cp3 = pltpu.async_copy(tcol_hbm.at[idxc], hc, s3)
            cp1.wait()
            cp2.wait()
            cp3.wait()

            @pl.loop(0, C)
            def _(c):
                v = ra[c, :] + rb[c, :]
                lr = jnp.maximum(v, 0.01 * v)
                w = jnp.exp(lr - gvec)
                m[c, pl.ds(DH, LANES)] = w
                for j in range(DH // LANES):
                    m[c, pl.ds(j * LANES, LANES)] = w * hc[c, pl.ds(j * LANES, LANES)]

            pltpu.sync_copy(m, acc.at[idxr], add=True)

        plsc.subcore_barrier()
        pltpu.sync_copy(acc.at[pl.ds(base_row, rows_per)],
                        out_hbm.at[ci, pl.ds(base_row, rows_per)])

    return sc_edges


def kernel(Z, edge_index, W_w, W_b, a_l, a_r):
    n = Z.shape[0]
    e = edge_index.shape[1]
    n_tab = n + 16
    n_acc = ((n // (NS * 8)) + 1) * NS * 8
    k_steps = -(-e // (NW * C))
    e_pad = NW * C * k_steps

    wt = W_w.T
    b2 = W_b.reshape(1, DH)
    eye8 = jnp.eye(NUM_HEADS, dtype=jnp.float32)
    alw = (a_l[0][:, :, None] * eye8[None, :, :]).reshape(DH, NUM_HEADS)
    arw = (a_r[0][:, :, None] * eye8[None, :, :]).reshape(DH, NUM_HEADS)
    tile8 = jnp.tile(eye8, (1, OUT_SIZE))

    row = edge_index[0].astype(jnp.int32)
    col = edge_index[1].astype(jnp.int32)
    padv = jnp.full((e_pad - e,), n, dtype=jnp.int32)
    rowp = jnp.concatenate([row, padv])
    colp = jnp.concatenate([col, padv])

    tcol, lrow, lcol, g = pl.pallas_call(
        _stage1_body,
        out_shape=(
            jax.ShapeDtypeStruct((n_tab, DH), jnp.float32),
            jax.ShapeDtypeStruct((n_tab, 2 * NUM_HEADS), jnp.float32),
            jax.ShapeDtypeStruct((n_tab, 2 * NUM_HEADS), jnp.float32),
            jax.ShapeDtypeStruct((1, 2 * NUM_HEADS), jnp.float32),
        ),
    )(Z, wt, b2, alw, arw)

    sc_edges = _make_sc_edge_kernel(n_tab, n_acc, e_pad, k_steps)
    s2 = sc_edges(rowp, colp, lrow, lcol, tcol, g)

    blk = 2000
    out128 = pl.pallas_call(
        _stage3_body,
        grid=(n // blk,),
        in_specs=[
            pl.BlockSpec((2, blk, ACC_W), lambda i: (0, i, 0)),
            pl.BlockSpec((NUM_HEADS, DH), lambda i: (0, 0)),
        ],
        out_specs=pl.BlockSpec((blk, DH), lambda i: (i, 0)),
        out_shape=jax.ShapeDtypeStruct((n, DH), jnp.float32),
    )(s2, tile8)

    return out128.reshape(n, OUT_SIZE, NUM_HEADS)


# speedup vs baseline: 1.4163x; 1.4163x over previous
import functools

import jax
import jax.numpy as jnp
from jax import lax
from jax.experimental import pallas as pl
from jax.experimental.pallas import tpu as pltpu
from jax.experimental.pallas import tpu_sc as plsc

NUM_HEADS = 8
OUT_SIZE = 16
DH = OUT_SIZE * NUM_HEADS

NC = 2
NS = 16
NW = NC * NS
LANES = 16
C = 96


def _stage1_body(z_ref, wt_ref, b_ref, alw_ref, arw_ref,
                 tcol_ref, lrow_ref, lcol_ref, g_ref):
    n = z_ref.shape[0]
    pad = tcol_ref.shape[0] - n
    h = jnp.dot(z_ref[...], wt_ref[...], preferred_element_type=jnp.float32,
                 precision=jax.lax.Precision.HIGHEST)
    h = h + b_ref[...]
    el = jnp.dot(h, alw_ref[...], preferred_element_type=jnp.float32,
                 precision=jax.lax.Precision.HIGHEST)
    er = jnp.dot(h, arw_ref[...], preferred_element_type=jnp.float32,
                 precision=jax.lax.Precision.HIGHEST)
    zpad_h = jnp.zeros((pad, DH), jnp.float32)
    zpad_e = jnp.zeros((pad, NUM_HEADS), jnp.float32)
    tcol_ref[...] = jnp.concatenate([h, zpad_h], axis=0)
    elp = jnp.concatenate([el, zpad_e], axis=0)
    erp = jnp.concatenate([er, zpad_e], axis=0)
    lrow_ref[...] = jnp.concatenate([elp, elp], axis=1)
    lcol_ref[...] = jnp.concatenate([erp, erp], axis=1)
    gmax = jnp.max(el, axis=0) + jnp.max(er, axis=0)
    g = jnp.maximum(gmax, 0.01 * gmax)
    g_ref[...] = jnp.concatenate([g, g]).reshape(1, 2 * NUM_HEADS)


def _stage3_body(sm_ref, sw_ref, t_ref, o_ref):
    msg = sm_ref[0] + sm_ref[1]
    den = (sw_ref[0] + sw_ref[1])[:, 0:NUM_HEADS]
    r = 1.0 / den
    r128 = jnp.dot(r, t_ref[...], preferred_element_type=jnp.float32,
                 precision=jax.lax.Precision.HIGHEST)
    o_ref[...] = msg * r128


def _make_sc_edge_kernel(n_tab, n_acc, e_pad, k_steps):
    rows_per = n_acc // NS
    wlanes = 2 * NUM_HEADS

    mesh = plsc.VectorSubcoreMesh(core_axis_name="c", subcore_axis_name="s")

    @functools.partial(
        pl.kernel,
        out_type=(jax.ShapeDtypeStruct((NC, n_acc, DH), jnp.float32),
                  jax.ShapeDtypeStruct((NC, n_acc, wlanes), jnp.float32)),
        mesh=mesh,
        compiler_params=pltpu.CompilerParams(use_tc_tiling_on_sc=False),
        scratch_types=[
            pltpu.VMEM((2, 2, C), jnp.int32),
            pltpu.VMEM((2, C), jnp.int32),
            pltpu.VMEM((2, C, wlanes), jnp.float32),
            pltpu.VMEM((2, C, wlanes), jnp.float32),
            pltpu.VMEM((2, C, DH), jnp.float32),
            pltpu.VMEM((2, C, wlanes), jnp.float32),
            pltpu.VMEM((wlanes,), jnp.float32),
            pltpu.VMEM_SHARED((n_acc, DH), jnp.float32),
            pltpu.VMEM_SHARED((n_acc, wlanes), jnp.float32),
            pltpu.SemaphoreType.DMA,
            pltpu.SemaphoreType.DMA,
            pltpu.SemaphoreType.DMA,
            pltpu.SemaphoreType.DMA,
            pltpu.SemaphoreType.DMA,
            pltpu.SemaphoreType.DMA,
        ],
    )
    def sc_edges(idx_hbm, lrow_hbm, lcol_hbm, tcol_hbm, g_hbm,
                 outm_hbm, outw_hbm, idxb, sidx, ra, rb, hc, wb, gv,
                 accm, accw, i0, i1, g0, g1, t0, t1):
        ci = lax.axis_index("c")
        si = lax.axis_index("s")
        wid = ci * NS + si
        isem = (i0, i1)
        gsem = (g0, g1)
        ssem = (t0, t1)

        zvec = jnp.zeros((LANES,), jnp.float32)

        @pl.loop(0, C)
        def _(r):
            wb[0, r, :] = zvec
            for j in range(DH // LANES):
                hc[0, r, pl.ds(j * LANES, LANES)] = zvec

        base_row = si * rows_per
        r0 = 0
        while r0 < rows_per:
            nr = min(rows_per - r0, C)
            pltpu.sync_copy(hc.at[0, pl.ds(0, nr)],
                            accm.at[pl.ds(base_row + r0, nr)])
            pltpu.sync_copy(wb.at[0, pl.ds(0, nr)],
                            accw.at[pl.ds(base_row + r0, nr)])
            r0 += nr

        pltpu.sync_copy(g_hbm.at[0], gv)
        gvec = gv[...]

        def issue_idx(t, p):
            pltpu.async_copy(idx_hbm.at[wid, t], idxb.at[p], isem[p])

        def drain_idx(p):
            pltpu.make_async_copy(idx_hbm.at[0, 0], idxb.at[p],
                                  isem[p]).wait()

        def issue_gathers(p):
            pltpu.async_copy(lrow_hbm.at[idxb.at[p, 0]], ra.at[p], gsem[p])
            pltpu.async_copy(lcol_hbm.at[idxb.at[p, 1]], rb.at[p], gsem[p])
            pltpu.async_copy(tcol_hbm.at[idxb.at[p, 1]], hc.at[p], gsem[p])

        def drain_gathers(p):
            pltpu.make_async_copy(lrow_hbm.at[pl.ds(0, C)], ra.at[p],
                                  gsem[p]).wait()
            pltpu.make_async_copy(lcol_hbm.at[pl.ds(0, C)], rb.at[p],
                                  gsem[p]).wait()
            pltpu.make_async_copy(tcol_hbm.at[pl.ds(0, C)], hc.at[p],
                                  gsem[p]).wait()

        def drain_scatter(p):
            pltpu.make_async_copy(hc.at[p], accm.at[pl.ds(0, C)],
                                  ssem[p]).wait()
            pltpu.make_async_copy(wb.at[p], accw.at[pl.ds(0, C)],
                                  ssem[p]).wait()

        plsc.subcore_barrier()
        issue_idx(0, 0)
        issue_idx(1, 1)
        drain_idx(0)
        issue_gathers(0)

        @pl.loop(0, k_steps // 2)
        def _(outer):
            for b in (0, 1):
                t = outer * 2 + b
                drain_gathers(b)

                @pl.when(t >= 1)
                def _():
                    drain_scatter(1 - b)

                for j in range(C // LANES):
                    sidx[b, pl.ds(j * LANES, LANES)] = (
                        idxb[b, 0, pl.ds(j * LANES, LANES)])

                @pl.when(t + 2 < k_steps)
                def _():
                    issue_idx(t + 2, b)

                @pl.when(t + 1 < k_steps)
                def _():
                    drain_idx(1 - b)
                    issue_gathers(1 - b)

                @pl.loop(0, C, unroll=4)
                def _(c):
                    v = ra[b, c, :] + rb[b, c, :]
                    lr = jnp.maximum(v, 0.01 * v)
                    w = jnp.exp(lr - gvec)
                    wb[b, c, :] = w
                    for j in range(DH // LANES):
                        hc[b, c, pl.ds(j * LANES, LANES)] = (
                            w * hc[b, c, pl.ds(j * LANES, LANES)])

                pltpu.async_copy(hc.at[b], accm.at[sidx.at[b]], ssem[b],
                                 add=True)
                pltpu.async_copy(wb.at[b], accw.at[sidx.at[b]], ssem[b],
                                 add=True)

        drain_scatter(1)
        plsc.subcore_barrier()
        pltpu.sync_copy(accm.at[pl.ds(base_row, rows_per)],
                        outm_hbm.at[ci, pl.ds(base_row, rows_per)])
        pltpu.sync_copy(accw.at[pl.ds(base_row, rows_per)],
                        outw_hbm.at[ci, pl.ds(base_row, rows_per)])

    return sc_edges


def kernel(Z, edge_index, W_w, W_b, a_l, a_r):
    n = Z.shape[0]
    e = edge_index.shape[1]
    n_tab = n + 16
    n_acc = ((n // (NS * 8)) + 1) * NS * 8
    k_steps = 2 * (-(-e // (NW * C * 2)))
    e_pad = NW * C * k_steps

    wt = W_w.T
    b2 = W_b.reshape(1, DH)
    eye8 = jnp.eye(NUM_HEADS, dtype=jnp.float32)
    alw = (a_l[0][:, :, None] * eye8[None, :, :]).reshape(DH, NUM_HEADS)
    arw = (a_r[0][:, :, None] * eye8[None, :, :]).reshape(DH, NUM_HEADS)
    tile8 = jnp.tile(eye8, (1, OUT_SIZE))

    row = edge_index[0].astype(jnp.int32)
    col = edge_index[1].astype(jnp.int32)
    padv = jnp.full((e_pad - e,), n, dtype=jnp.int32)
    rowp = jnp.concatenate([row, padv]).reshape(NW, k_steps, 1, C)
    colp = jnp.concatenate([col, padv]).reshape(NW, k_steps, 1, C)
    idx_blocks = jnp.concatenate([rowp, colp], axis=2)

    tcol, lrow, lcol, g = pl.pallas_call(
        _stage1_body,
        out_shape=(
            jax.ShapeDtypeStruct((n_tab, DH), jnp.float32),
            jax.ShapeDtypeStruct((n_tab, 2 * NUM_HEADS), jnp.float32),
            jax.ShapeDtypeStruct((n_tab, 2 * NUM_HEADS), jnp.float32),
            jax.ShapeDtypeStruct((1, 2 * NUM_HEADS), jnp.float32),
        ),
    )(Z, wt, b2, alw, arw)

    sc_edges = _make_sc_edge_kernel(n_tab, n_acc, e_pad, k_steps)
    s2m, s2w = sc_edges(idx_blocks, lrow, lcol, tcol, g)

    blk = 2000
    out128 = pl.pallas_call(
        _stage3_body,
        grid=(n // blk,),
        in_specs=[
            pl.BlockSpec((2, blk, DH), lambda i: (0, i, 0)),
            pl.BlockSpec((2, blk, 2 * NUM_HEADS), lambda i: (0, i, 0)),
            pl.BlockSpec((NUM_HEADS, DH), lambda i: (0, 0)),
        ],
        out_specs=pl.BlockSpec((blk, DH), lambda i: (i, 0)),
        out_shape=jax.ShapeDtypeStruct((n, DH), jnp.float32),
    )(s2m, s2w, tile8)

    return out128.reshape(n, OUT_SIZE, NUM_HEADS)


# speedup vs baseline: 1.5529x; 1.0964x over previous
import functools

import jax
import jax.numpy as jnp
from jax import lax
from jax.experimental import pallas as pl
from jax.experimental.pallas import tpu as pltpu
from jax.experimental.pallas import tpu_sc as plsc

NUM_HEADS = 8
OUT_SIZE = 16
DH = OUT_SIZE * NUM_HEADS

NC = 2
NS = 16
NW = NC * NS
LANES = 16
C = 112
ROW_W = DH + LANES


def _stage1_body(z_ref, wt_ref, b_ref, alw_ref, arw_ref,
                 tcol_ref, lrow_ref, g_ref):
    n = z_ref.shape[0]
    pad = tcol_ref.shape[0] - n
    h = jnp.dot(z_ref[...], wt_ref[...], preferred_element_type=jnp.float32,
                 precision=jax.lax.Precision.HIGHEST)
    h = h + b_ref[...]
    el = jnp.dot(h, alw_ref[...], preferred_element_type=jnp.float32,
                 precision=jax.lax.Precision.HIGHEST)
    er = jnp.dot(h, arw_ref[...], preferred_element_type=jnp.float32,
                 precision=jax.lax.Precision.HIGHEST)
    zpad_h = jnp.zeros((pad, DH), jnp.float32)
    zpad_e = jnp.zeros((pad, NUM_HEADS), jnp.float32)
    hp = jnp.concatenate([h, zpad_h], axis=0)
    elp = jnp.concatenate([el, zpad_e], axis=0)
    erp = jnp.concatenate([er, zpad_e], axis=0)
    tcol_ref[...] = jnp.concatenate([hp, erp, erp], axis=1)
    lrow_ref[...] = jnp.concatenate([elp, elp], axis=1)
    gmax = jnp.max(el, axis=0) + jnp.max(er, axis=0)
    g = jnp.maximum(gmax, 0.01 * gmax)
    g_ref[...] = jnp.concatenate([g, g]).reshape(1, 2 * NUM_HEADS)


def _stage3_body(s2_ref, t_ref, o_ref):
    s = s2_ref[0] + s2_ref[1]
    msg = s[:, 0:DH]
    den = s[:, DH:DH + NUM_HEADS]
    r = 1.0 / den
    r128 = jnp.dot(r, t_ref[...], preferred_element_type=jnp.float32,
                 precision=jax.lax.Precision.HIGHEST)
    o_ref[...] = msg * r128


def _make_sc_edge_kernel(n_tab, n_acc, e_pad, k_steps):
    rows_per = n_acc // NS

    mesh = plsc.VectorSubcoreMesh(core_axis_name="c", subcore_axis_name="s")

    @functools.partial(
        pl.kernel,
        out_type=jax.ShapeDtypeStruct((NC, n_acc, ROW_W), jnp.float32),
        mesh=mesh,
        compiler_params=pltpu.CompilerParams(use_tc_tiling_on_sc=False),
        scratch_types=[
            pltpu.VMEM((2, 2, C), jnp.int32),
            pltpu.VMEM((2, C), jnp.int32),
            pltpu.VMEM((2, C, 2 * NUM_HEADS), jnp.float32),
            pltpu.VMEM((2, C, ROW_W), jnp.float32),
            pltpu.VMEM((2 * NUM_HEADS,), jnp.float32),
            pltpu.VMEM_SHARED((n_acc, ROW_W), jnp.float32),
            pltpu.SemaphoreType.DMA,
            pltpu.SemaphoreType.DMA,
            pltpu.SemaphoreType.DMA,
            pltpu.SemaphoreType.DMA,
            pltpu.SemaphoreType.DMA,
            pltpu.SemaphoreType.DMA,
        ],
    )
    def sc_edges(idx_hbm, lrow_hbm, te_hbm, g_hbm,
                 out_hbm, idxb, sidx, ra, hc, gv,
                 acc, i0, i1, g0, g1, t0, t1):
        ci = lax.axis_index("c")
        si = lax.axis_index("s")
        wid = ci * NS + si
        isem = (i0, i1)
        gsem = (g0, g1)
        ssem = (t0, t1)

        zvec = jnp.zeros((LANES,), jnp.float32)

        @pl.loop(0, C)
        def _(r):
            for j in range(ROW_W // LANES):
                hc[0, r, pl.ds(j * LANES, LANES)] = zvec

        base_row = si * rows_per
        r0 = 0
        while r0 < rows_per:
            nr = min(rows_per - r0, C)
            pltpu.sync_copy(hc.at[0, pl.ds(0, nr)],
                            acc.at[pl.ds(base_row + r0, nr)])
            r0 += nr

        pltpu.sync_copy(g_hbm.at[0], gv)
        gvec = gv[...]

        def issue_idx(t, p):
            pltpu.async_copy(idx_hbm.at[wid, t], idxb.at[p], isem[p])

        def drain_idx(p):
            pltpu.make_async_copy(idx_hbm.at[0, 0], idxb.at[p],
                                  isem[p]).wait()

        def issue_gathers(p):
            pltpu.async_copy(lrow_hbm.at[idxb.at[p, 0]], ra.at[p], gsem[p])
            pltpu.async_copy(te_hbm.at[idxb.at[p, 1]], hc.at[p], gsem[p])

        def drain_gathers(p):
            pltpu.make_async_copy(lrow_hbm.at[pl.ds(0, C)], ra.at[p],
                                  gsem[p]).wait()
            pltpu.make_async_copy(te_hbm.at[pl.ds(0, C)], hc.at[p],
                                  gsem[p]).wait()

        def drain_scatter(p):
            pltpu.make_async_copy(hc.at[p], acc.at[pl.ds(0, C)],
                                  ssem[p]).wait()

        plsc.subcore_barrier()
        issue_idx(0, 0)
        issue_idx(1, 1)
        drain_idx(0)
        issue_gathers(0)

        @pl.loop(0, k_steps // 2)
        def _(outer):
            for b in (0, 1):
                t = outer * 2 + b
                drain_gathers(b)

                @pl.when(t >= 1)
                def _():
                    drain_scatter(1 - b)

                for j in range(C // LANES):
                    sidx[b, pl.ds(j * LANES, LANES)] = (
                        idxb[b, 0, pl.ds(j * LANES, LANES)])

                @pl.when(t + 2 < k_steps)
                def _():
                    issue_idx(t + 2, b)

                @pl.when(t + 1 < k_steps)
                def _():
                    drain_idx(1 - b)
                    issue_gathers(1 - b)

                @pl.loop(0, C, unroll=4)
                def _(c):
                    v = ra[b, c, :] + hc[b, c, pl.ds(DH, LANES)]
                    lr = jnp.maximum(v, 0.01 * v)
                    w = jnp.exp(lr - gvec)
                    hc[b, c, pl.ds(DH, LANES)] = w
                    for j in range(DH // LANES):
                        hc[b, c, pl.ds(j * LANES, LANES)] = (
                            w * hc[b, c, pl.ds(j * LANES, LANES)])

                pltpu.async_copy(hc.at[b], acc.at[sidx.at[b]], ssem[b],
                                 add=True)

        drain_scatter(1)
        plsc.subcore_barrier()
        pltpu.sync_copy(acc.at[pl.ds(base_row, rows_per)],
                        out_hbm.at[ci, pl.ds(base_row, rows_per)])

    return sc_edges


def kernel(Z, edge_index, W_w, W_b, a_l, a_r):
    n = Z.shape[0]
    e = edge_index.shape[1]
    n_tab = n + 16
    n_acc = ((n // (NS * 8)) + 1) * NS * 8
    k_steps = 2 * (-(-e // (NW * C * 2)))
    e_pad = NW * C * k_steps

    wt = W_w.T
    b2 = W_b.reshape(1, DH)
    eye8 = jnp.eye(NUM_HEADS, dtype=jnp.float32)
    alw = (a_l[0][:, :, None] * eye8[None, :, :]).reshape(DH, NUM_HEADS)
    arw = (a_r[0][:, :, None] * eye8[None, :, :]).reshape(DH, NUM_HEADS)
    tile8 = jnp.tile(eye8, (1, OUT_SIZE))

    row = edge_index[0].astype(jnp.int32)
    col = edge_index[1].astype(jnp.int32)
    padv = jnp.full((e_pad - e,), n, dtype=jnp.int32)
    rowp = jnp.concatenate([row, padv]).reshape(NW, k_steps, 1, C)
    colp = jnp.concatenate([col, padv]).reshape(NW, k_steps, 1, C)
    idx_blocks = jnp.concatenate([rowp, colp], axis=2)

    te, lrow, g = pl.pallas_call(
        _stage1_body,
        out_shape=(
            jax.ShapeDtypeStruct((n_tab, ROW_W), jnp.float32),
            jax.ShapeDtypeStruct((n_tab, 2 * NUM_HEADS), jnp.float32),
            jax.ShapeDtypeStruct((1, 2 * NUM_HEADS), jnp.float32),
        ),
    )(Z, wt, b2, alw, arw)

    sc_edges = _make_sc_edge_kernel(n_tab, n_acc, e_pad, k_steps)
    s2 = sc_edges(idx_blocks, lrow, te, g)

    blk = 2000
    out128 = pl.pallas_call(
        _stage3_body,
        grid=(n // blk,),
        in_specs=[
            pl.BlockSpec((2, blk, ROW_W), lambda i: (0, i, 0)),
            pl.BlockSpec((NUM_HEADS, DH), lambda i: (0, 0)),
        ],
        out_specs=pl.BlockSpec((blk, DH), lambda i: (i, 0)),
        out_shape=jax.ShapeDtypeStruct((n, DH), jnp.float32),
    )(s2, tile8)

    return out128.reshape(n, OUT_SIZE, NUM_HEADS)


# speedup vs baseline: 2.4549x; 1.5808x over previous
import functools

import jax
import jax.numpy as jnp
from jax import lax
from jax.experimental import pallas as pl
from jax.experimental.pallas import tpu as pltpu
from jax.experimental.pallas import tpu_sc as plsc

NUM_HEADS = 8
OUT_SIZE = 16
DH = OUT_SIZE * NUM_HEADS

NC = 2
NS = 16
NW = NC * NS
LANES = 16
C = 112
ROW_W = DH + LANES


def _stage1_body(z_ref, wt_ref, b_ref, alw_ref, arw_ref,
                 tcol_ref, lrow_ref, g_ref):
    n = z_ref.shape[0]
    pad = tcol_ref.shape[0] - n
    h = jnp.dot(z_ref[...], wt_ref[...], preferred_element_type=jnp.float32,
                 precision=jax.lax.Precision.HIGHEST)
    h = h + b_ref[...]
    el = jnp.dot(h, alw_ref[...], preferred_element_type=jnp.float32,
                 precision=jax.lax.Precision.HIGHEST)
    er = jnp.dot(h, arw_ref[...], preferred_element_type=jnp.float32,
                 precision=jax.lax.Precision.HIGHEST)
    zpad_h = jnp.zeros((pad, DH), jnp.float32)
    zpad_e = jnp.zeros((pad, NUM_HEADS), jnp.float32)
    hp = jnp.concatenate([h, zpad_h], axis=0)
    elp = jnp.concatenate([el, zpad_e], axis=0)
    erp = jnp.concatenate([er, zpad_e], axis=0)
    tcol_ref[...] = jnp.concatenate([hp, erp, erp], axis=1)
    lrow_ref[...] = jnp.concatenate([elp, elp], axis=1)
    gmax = jnp.max(el, axis=0) + jnp.max(er, axis=0)
    g = jnp.maximum(gmax, 0.01 * gmax)
    g_ref[...] = jnp.concatenate([g, g]).reshape(1, 2 * NUM_HEADS)


def _stage3_body(s2_ref, t_ref, o_ref):
    s = s2_ref[0] + s2_ref[1]
    msg = s[:, 0:DH]
    den = s[:, DH:DH + NUM_HEADS]
    r = 1.0 / den
    r128 = jnp.dot(r, t_ref[...], preferred_element_type=jnp.float32,
                 precision=jax.lax.Precision.HIGHEST)
    o_ref[...] = msg * r128


def _make_sc_edge_kernel(n_tab, n_acc, e_pad, k_steps):
    rows_per = n_acc // NS

    mesh = plsc.VectorSubcoreMesh(core_axis_name="c", subcore_axis_name="s")

    @functools.partial(
        pl.kernel,
        out_type=jax.ShapeDtypeStruct((NC, n_acc, ROW_W), jnp.float32),
        mesh=mesh,
        compiler_params=pltpu.CompilerParams(use_tc_tiling_on_sc=False),
        scratch_types=[
            pltpu.VMEM((2, 2, C), jnp.int32),
            pltpu.VMEM((2, C), jnp.int32),
            pltpu.VMEM((2, C, 2 * NUM_HEADS), jnp.float32),
            pltpu.VMEM((2, C, ROW_W), jnp.float32),
            pltpu.VMEM((2 * NUM_HEADS,), jnp.float32),
            pltpu.VMEM_SHARED((n_acc, ROW_W), jnp.float32),
            pltpu.SemaphoreType.DMA,
            pltpu.SemaphoreType.DMA,
            pltpu.SemaphoreType.DMA,
            pltpu.SemaphoreType.DMA,
            pltpu.SemaphoreType.DMA,
            pltpu.SemaphoreType.DMA,
        ],
    )
    def sc_edges(rowi_hbm, coli_hbm, lrow_hbm, te_hbm, g_hbm,
                 out_hbm, idxb, sidx, ra, hc, gv,
                 acc, i0, i1, g0, g1, t0, t1):
        ci = lax.axis_index("c")
        si = lax.axis_index("s")
        wid = ci * NS + si
        isem = (i0, i1)
        gsem = (g0, g1)
        ssem = (t0, t1)

        zvec = jnp.zeros((LANES,), jnp.float32)

        @pl.loop(0, C)
        def _(r):
            for j in range(ROW_W // LANES):
                hc[0, r, pl.ds(j * LANES, LANES)] = zvec

        base_row = si * rows_per
        r0 = 0
        while r0 < rows_per:
            nr = min(rows_per - r0, C)
            pltpu.sync_copy(hc.at[0, pl.ds(0, nr)],
                            acc.at[pl.ds(base_row + r0, nr)])
            r0 += nr

        pltpu.sync_copy(g_hbm.at[0], gv)
        gvec = gv[...]

        def issue_idx(t, p):
            pltpu.async_copy(rowi_hbm.at[wid, t], idxb.at[p, 0], isem[p])
            pltpu.async_copy(coli_hbm.at[wid, t], idxb.at[p, 1], isem[p])

        def drain_idx(p):
            pltpu.make_async_copy(rowi_hbm.at[0, 0], idxb.at[p, 0],
                                  isem[p]).wait()
            pltpu.make_async_copy(coli_hbm.at[0, 0], idxb.at[p, 1],
                                  isem[p]).wait()

        def issue_gathers(p):
            pltpu.async_copy(lrow_hbm.at[idxb.at[p, 0]], ra.at[p], gsem[p])
            pltpu.async_copy(te_hbm.at[idxb.at[p, 1]], hc.at[p], gsem[p])

        def drain_gathers(p):
            pltpu.make_async_copy(lrow_hbm.at[pl.ds(0, C)], ra.at[p],
                                  gsem[p]).wait()
            pltpu.make_async_copy(te_hbm.at[pl.ds(0, C)], hc.at[p],
                                  gsem[p]).wait()

        def drain_scatter(p):
            pltpu.make_async_copy(hc.at[p], acc.at[pl.ds(0, C)],
                                  ssem[p]).wait()

        plsc.subcore_barrier()
        issue_idx(0, 0)
        issue_idx(1, 1)
        drain_idx(0)
        issue_gathers(0)

        @pl.loop(0, k_steps // 2)
        def _(outer):
            for b in (0, 1):
                t = outer * 2 + b
                drain_gathers(b)

                @pl.when(t >= 1)
                def _():
                    drain_scatter(1 - b)

                for j in range(C // LANES):
                    sidx[b, pl.ds(j * LANES, LANES)] = (
                        idxb[b, 0, pl.ds(j * LANES, LANES)])

                @pl.when(t + 2 < k_steps)
                def _():
                    issue_idx(t + 2, b)

                @pl.when(t + 1 < k_steps)
                def _():
                    drain_idx(1 - b)
                    issue_gathers(1 - b)

                @pl.loop(0, C, unroll=4)
                def _(c):
                    v = ra[b, c, :] + hc[b, c, pl.ds(DH, LANES)]
                    lr = jnp.maximum(v, 0.01 * v)
                    w = jnp.exp(lr - gvec)
                    hc[b, c, pl.ds(DH, LANES)] = w
                    for j in range(DH // LANES):
                        hc[b, c, pl.ds(j * LANES, LANES)] = (
                            w * hc[b, c, pl.ds(j * LANES, LANES)])

                pltpu.async_copy(hc.at[b], acc.at[sidx.at[b]], ssem[b],
                                 add=True)

        drain_scatter(1)
        plsc.subcore_barrier()
        pltpu.sync_copy(acc.at[pl.ds(base_row, rows_per)],
                        out_hbm.at[ci, pl.ds(base_row, rows_per)])

    return sc_edges


def kernel(Z, edge_index, W_w, W_b, a_l, a_r):
    n = Z.shape[0]
    e = edge_index.shape[1]
    n_tab = n + 16
    n_acc = ((n // (NS * 8)) + 1) * NS * 8
    k_steps = 2 * (-(-e // (NW * C * 2)))
    e_pad = NW * C * k_steps

    wt = W_w.T
    b2 = W_b.reshape(1, DH)
    eye8 = jnp.eye(NUM_HEADS, dtype=jnp.float32)
    alw = (a_l[0][:, :, None] * eye8[None, :, :]).reshape(DH, NUM_HEADS)
    arw = (a_r[0][:, :, None] * eye8[None, :, :]).reshape(DH, NUM_HEADS)
    tile8 = jnp.tile(eye8, (1, OUT_SIZE))

    row = edge_index[0].astype(jnp.int32)
    col = edge_index[1].astype(jnp.int32)
    ar = jnp.arange(e_pad - e, dtype=jnp.int32)
    pad_rows = n + ar % (n_acc - n)
    pad_cols = n + ar % (n_tab - n)
    rowp = jnp.concatenate([row, pad_rows]).reshape(NW, k_steps, C)
    colp = jnp.concatenate([col, pad_cols]).reshape(NW, k_steps, C)

    te, lrow, g = pl.pallas_call(
        _stage1_body,
        out_shape=(
            jax.ShapeDtypeStruct((n_tab, ROW_W), jnp.float32),
            jax.ShapeDtypeStruct((n_tab, 2 * NUM_HEADS), jnp.float32),
            jax.ShapeDtypeStruct((1, 2 * NUM_HEADS), jnp.float32),
        ),
    )(Z, wt, b2, alw, arw)

    sc_edges = _make_sc_edge_kernel(n_tab, n_acc, e_pad, k_steps)
    s2 = sc_edges(rowp, colp, lrow, te, g)

    blk = 2000
    out128 = pl.pallas_call(
        _stage3_body,
        grid=(n // blk,),
        in_specs=[
            pl.BlockSpec((2, blk, ROW_W), lambda i: (0, i, 0)),
            pl.BlockSpec((NUM_HEADS, DH), lambda i: (0, 0)),
        ],
        out_specs=pl.BlockSpec((blk, DH), lambda i: (i, 0)),
        out_shape=jax.ShapeDtypeStruct((n, DH), jnp.float32),
    )(s2, tile8)

    return out128.reshape(n, OUT_SIZE, NUM_HEADS)
